# baseline (speedup 1.0000x reference)
import functools

import jax
import jax.numpy as jnp
from jax import lax
from jax.experimental import pallas as pl
from jax.experimental.pallas import tpu as pltpu
from jax.experimental.pallas import tpu_sc as plsc

D_MODEL = 768
BATCH = 4
SEQ = 2048
NB = BATCH * SEQ
NW = 32
S_PER_W = SEQ // NW
K = 32
N_CHUNKS = BATCH * S_PER_W // K
LANES = 16
JCOLS = D_MODEL // LANES


def _make_embed():
    mesh = plsc.VectorSubcoreMesh(core_axis_name="c", subcore_axis_name="s")

    @functools.partial(
        pl.kernel,
        mesh=mesh,
        out_type=jax.ShapeDtypeStruct((NB, D_MODEL), jnp.float32),
        scratch_types=[
            pltpu.VMEM((BATCH * S_PER_W,), jnp.int32),
            pltpu.VMEM((S_PER_W, D_MODEL), jnp.float32),
            pltpu.VMEM((2, K, D_MODEL), jnp.float32),
            pltpu.SemaphoreType.DMA,
            pltpu.SemaphoreType.DMA,
            pltpu.SemaphoreType.DMA,
            pltpu.SemaphoreType.DMA,
            pltpu.SemaphoreType.DMA,
        ],
    )
    def embed(x_hbm, table_hbm, pos_hbm, out_hbm,
              idx_v, pos_v, rows_v, isem, psem,
              gsem0, gsem1, ssem):
        wid = lax.axis_index("s") * 2 + lax.axis_index("c")
        s_base = pl.multiple_of(wid * S_PER_W, S_PER_W)

        idx_copies = [
            pltpu.async_copy(
                x_hbm.at[pl.ds(b * SEQ + s_base, S_PER_W)],
                idx_v.at[pl.ds(b * S_PER_W, S_PER_W)], isem)
            for b in range(BATCH)
        ]
        pos_copy = pltpu.async_copy(
            pos_hbm.at[pl.ds(s_base, S_PER_W)], pos_v, psem)
        for cp in idx_copies:
            cp.wait()

        gsems = [gsem0, gsem1]
        rows_bufs = [rows_v.at[0], rows_v.at[1]]

        def start_gather(c):
            return pltpu.async_copy(
                table_hbm.at[idx_v.at[pl.ds(c * K, K)]],
                rows_bufs[c % 2], gsems[c % 2])

        def start_store(c):
            b, h = divmod(c, N_CHUNKS // BATCH)
            out_row = pl.multiple_of(b * SEQ + s_base + h * K, 8)
            return pltpu.async_copy(
                rows_bufs[c % 2], out_hbm.at[pl.ds(out_row, K)], ssem)

        gathers = [None] * N_CHUNKS
        stores = [None] * N_CHUNKS
        gathers[0] = start_gather(0)
        pos_copy.wait()
        for c in range(N_CHUNKS):
            if c + 1 < N_CHUNKS:
                if c >= 1:
                    stores[c - 1].wait()
                gathers[c + 1] = start_gather(c + 1)
            gathers[c].wait()
            h = c % (N_CHUNKS // BATCH)
            buf = rows_bufs[c % 2]

            def row(r, _, h=h, buf=buf):
                for j in range(JCOLS):
                    sl = pl.ds(j * LANES, LANES)
                    buf[r, sl] = buf[r, sl] + pos_v[h * K + r, sl]
                return _

            lax.fori_loop(0, K, row, 0)
            stores[c] = start_store(c)
        stores[N_CHUNKS - 2].wait()
        stores[N_CHUNKS - 1].wait()

    return embed


_embed = _make_embed()


def kernel(x, token_table, pos_emb):
    x_flat = x.reshape(NB)
    pos = pos_emb.reshape(pos_emb.shape[1], D_MODEL)
    out = _embed(x_flat, token_table, pos)
    return out.reshape(BATCH, SEQ, D_MODEL)


# speedup vs baseline: 1.2420x; 1.2420x over previous
import functools

import jax
import jax.numpy as jnp
from jax import lax
from jax.experimental import pallas as pl
from jax.experimental.pallas import tpu as pltpu
from jax.experimental.pallas import tpu_sc as plsc

D_MODEL = 768
BATCH = 4
SEQ = 2048
NB = BATCH * SEQ
NW = 32
S_PER_W = SEQ // NW
K = 32
N_CHUNKS = BATCH * S_PER_W // K
LANES = 16
JCOLS = D_MODEL // LANES


def _make_embed():
    mesh = plsc.VectorSubcoreMesh(core_axis_name="c", subcore_axis_name="s")

    @functools.partial(
        pl.kernel,
        mesh=mesh,
        out_type=jax.ShapeDtypeStruct((NB, D_MODEL), jnp.float32),
        scratch_types=[
            pltpu.VMEM((BATCH * S_PER_W,), jnp.int32),
            pltpu.VMEM((S_PER_W, D_MODEL), jnp.float32),
            pltpu.VMEM((2, K, D_MODEL), jnp.float32),
            pltpu.SemaphoreType.DMA,
            pltpu.SemaphoreType.DMA,
            pltpu.SemaphoreType.DMA,
            pltpu.SemaphoreType.DMA,
            pltpu.SemaphoreType.DMA,
        ],
    )
    def embed(x_hbm, table_hbm, pos_hbm, out_hbm,
              idx_v, pos_v, rows_v, isem, psem,
              gsem0, gsem1, ssem):
        wid = lax.axis_index("s") * 2 + lax.axis_index("c")
        s_base = pl.multiple_of(wid * S_PER_W, S_PER_W)

        idx_copies = [
            pltpu.async_copy(
                x_hbm.at[pl.ds(b * SEQ + s_base, S_PER_W)],
                idx_v.at[pl.ds(b * S_PER_W, S_PER_W)], isem)
            for b in range(BATCH)
        ]
        pos_copy = pltpu.async_copy(
            pos_hbm.at[pl.ds(s_base, S_PER_W)], pos_v, psem)
        for cp in idx_copies:
            cp.wait()

        gsems = [gsem0, gsem1]
        rows_bufs = [rows_v.at[0], rows_v.at[1]]

        def start_gather(c):
            return pltpu.async_copy(
                table_hbm.at[idx_v.at[pl.ds(c * K, K)]],
                rows_bufs[c % 2], gsems[c % 2])

        def start_store(c):
            b, h = divmod(c, N_CHUNKS // BATCH)
            out_row = pl.multiple_of(b * SEQ + s_base + h * K, 8)
            return pltpu.async_copy(
                rows_bufs[c % 2], out_hbm.at[pl.ds(out_row, K)], ssem)

        gathers = [None] * N_CHUNKS
        stores = [None] * N_CHUNKS
        gathers[0] = start_gather(0)
        pos_copy.wait()
        for c in range(N_CHUNKS):
            if c + 1 < N_CHUNKS:
                if c >= 1:
                    stores[c - 1].wait()
                gathers[c + 1] = start_gather(c + 1)
            gathers[c].wait()
            h = c % (N_CHUNKS // BATCH)
            buf = rows_bufs[c % 2]

            @plsc.parallel_loop(0, K, 1, unroll=2)
            def row(r, h=h, buf=buf):
                for j in range(JCOLS):
                    sl = pl.ds(j * LANES, LANES)
                    buf[r, sl] = buf[r, sl] + pos_v[h * K + r, sl]

            stores[c] = start_store(c)
        stores[N_CHUNKS - 2].wait()
        stores[N_CHUNKS - 1].wait()

    return embed


_embed = _make_embed()


def kernel(x, token_table, pos_emb):
    x_flat = x.reshape(NB)
    pos = pos_emb.reshape(pos_emb.shape[1], D_MODEL)
    out = _embed(x_flat, token_table, pos)
    return out.reshape(BATCH, SEQ, D_MODEL)


# speedup vs baseline: 1.2828x; 1.0329x over previous
import functools

import jax
import jax.numpy as jnp
from jax import lax
from jax.experimental import pallas as pl
from jax.experimental.pallas import tpu as pltpu
from jax.experimental.pallas import tpu_sc as plsc

D_MODEL = 768
BATCH = 4
SEQ = 2048
NB = BATCH * SEQ
NW = 32
S_PER_W = SEQ // NW
K = 32
N_CHUNKS = BATCH * S_PER_W // K
LANES = 16
JCOLS = D_MODEL // LANES


def _make_embed():
    mesh = plsc.VectorSubcoreMesh(core_axis_name="c", subcore_axis_name="s")

    @functools.partial(
        pl.kernel,
        mesh=mesh,
        out_type=jax.ShapeDtypeStruct((NB, D_MODEL), jnp.float32),
        scratch_types=[
            pltpu.VMEM((BATCH * S_PER_W,), jnp.int32),
            pltpu.VMEM((S_PER_W, D_MODEL), jnp.float32),
            pltpu.VMEM((2, K, D_MODEL), jnp.float32),
            pltpu.SemaphoreType.DMA,
            pltpu.SemaphoreType.DMA,
            pltpu.SemaphoreType.DMA,
            pltpu.SemaphoreType.DMA,
            pltpu.SemaphoreType.DMA,
        ],
    )
    def embed(x_hbm, table_hbm, pos_hbm, out_hbm,
              idx_v, pos_v, rows_v, isem, psem,
              gsem0, gsem1, ssem):
        wid = lax.axis_index("s") * 2 + lax.axis_index("c")
        s_base = pl.multiple_of(wid * S_PER_W, S_PER_W)

        idx_copies = [
            pltpu.async_copy(
                x_hbm.at[pl.ds(b * SEQ + s_base, S_PER_W)],
                idx_v.at[pl.ds(b * S_PER_W, S_PER_W)], isem)
            for b in range(BATCH)
        ]
        pos_copy = pltpu.async_copy(
            pos_hbm.at[pl.ds(s_base, S_PER_W)], pos_v, psem)
        for cp in idx_copies:
            cp.wait()

        gsems = [gsem0, gsem1]
        rows_bufs = [rows_v.at[0], rows_v.at[1]]

        def start_gather(c):
            return pltpu.async_copy(
                table_hbm.at[idx_v.at[pl.ds(c * K, K)]],
                rows_bufs[c % 2], gsems[c % 2])

        def start_store(c):
            b, h = divmod(c, N_CHUNKS // BATCH)
            out_row = pl.multiple_of(b * SEQ + s_base + h * K, 8)
            return pltpu.async_copy(
                rows_bufs[c % 2], out_hbm.at[pl.ds(out_row, K)], ssem)

        gathers = [None] * N_CHUNKS
        stores = [None] * N_CHUNKS
        gathers[0] = start_gather(0)
        pos_copy.wait()
        for c in range(N_CHUNKS):
            if c + 1 < N_CHUNKS:
                if c >= 1:
                    stores[c - 1].wait()
                gathers[c + 1] = start_gather(c + 1)
            gathers[c].wait()
            h = c % (N_CHUNKS // BATCH)
            buf = rows_bufs[c % 2]

            @plsc.parallel_loop(0, K, 1, unroll=2)
            def row(r, h=h, buf=buf):
                for j in range(JCOLS):
                    sl = pl.ds(j * LANES, LANES)
                    plsc.addupdate(buf.at[r, sl], pos_v[h * K + r, sl])

            stores[c] = start_store(c)
        stores[N_CHUNKS - 2].wait()
        stores[N_CHUNKS - 1].wait()

    return embed


_embed = _make_embed()


def kernel(x, token_table, pos_emb):
    x_flat = x.reshape(NB)
    pos = pos_emb.reshape(pos_emb.shape[1], D_MODEL)
    out = _embed(x_flat, token_table, pos)
    return out.reshape(BATCH, SEQ, D_MODEL)


# speedup vs baseline: 1.3567x; 1.0576x over previous
import functools

import jax
import jax.numpy as jnp
from jax import lax
from jax.experimental import pallas as pl
from jax.experimental.pallas import tpu as pltpu
from jax.experimental.pallas import tpu_sc as plsc

D_MODEL = 768
BATCH = 4
SEQ = 2048
NB = BATCH * SEQ
NW = 32
S_PER_W = SEQ // NW
K = 32
N_CHUNKS = BATCH * S_PER_W // K
NBUF = 3
LANES = 16
JCOLS = D_MODEL // LANES


def _make_embed():
    mesh = plsc.VectorSubcoreMesh(core_axis_name="c", subcore_axis_name="s")

    @functools.partial(
        pl.kernel,
        mesh=mesh,
        out_type=jax.ShapeDtypeStruct((NB, D_MODEL), jnp.float32),
        scratch_types=[
            pltpu.VMEM((BATCH * S_PER_W,), jnp.int32),
            pltpu.VMEM((S_PER_W, D_MODEL), jnp.float32),
            pltpu.VMEM((NBUF, K, D_MODEL), jnp.float32),
            pltpu.SemaphoreType.DMA,
            pltpu.SemaphoreType.DMA,
            pltpu.SemaphoreType.DMA,
            pltpu.SemaphoreType.DMA,
        ],
    )
    def embed(x_hbm, table_hbm, pos_hbm, out_hbm,
              idx_v, pos_v, rows_v, isem, psem, gsem, ssem):
        wid = lax.axis_index("s") * 2 + lax.axis_index("c")
        s_base = pl.multiple_of(wid * S_PER_W, S_PER_W)

        idx_copies = [
            pltpu.async_copy(
                x_hbm.at[pl.ds(b * SEQ + s_base, S_PER_W)],
                idx_v.at[pl.ds(b * S_PER_W, S_PER_W)], isem)
            for b in range(BATCH)
        ]
        pos_copy = pltpu.async_copy(
            pos_hbm.at[pl.ds(s_base, S_PER_W)], pos_v, psem)
        for cp in idx_copies:
            cp.wait()

        rows_bufs = [rows_v.at[p] for p in range(NBUF)]

        def start_gather(c):
            return pltpu.async_copy(
                table_hbm.at[idx_v.at[pl.ds(c * K, K)]],
                rows_bufs[c % NBUF], gsem)

        def start_store(c):
            b, h = divmod(c, N_CHUNKS // BATCH)
            out_row = pl.multiple_of(b * SEQ + s_base + h * K, 8)
            return pltpu.async_copy(
                rows_bufs[c % NBUF], out_hbm.at[pl.ds(out_row, K)], ssem)

        gathers = [None] * N_CHUNKS
        stores = [None] * N_CHUNKS
        gathers[0] = start_gather(0)
        gathers[1] = start_gather(1)
        pos_copy.wait()
        for c in range(N_CHUNKS):
            if c + 1 < N_CHUNKS and gathers[c + 1] is None:
                if c + 1 >= NBUF:
                    stores[c + 1 - NBUF].wait()
                gathers[c + 1] = start_gather(c + 1)
            gathers[c].wait()
            h = c % (N_CHUNKS // BATCH)
            buf = rows_bufs[c % NBUF]

            @plsc.parallel_loop(0, K, 1, unroll=2)
            def row(r, h=h, buf=buf):
                for j in range(JCOLS):
                    sl = pl.ds(j * LANES, LANES)
                    plsc.addupdate(buf.at[r, sl], pos_v[h * K + r, sl])

            stores[c] = start_store(c)
        for c in range(N_CHUNKS - NBUF, N_CHUNKS):
            stores[c].wait()

    return embed


_embed = _make_embed()


def kernel(x, token_table, pos_emb):
    x_flat = x.reshape(NB)
    pos = pos_emb.reshape(pos_emb.shape[1], D_MODEL)
    out = _embed(x_flat, token_table, pos)
    return out.reshape(BATCH, SEQ, D_MODEL)


# speedup vs baseline: 1.6554x; 1.2201x over previous
import functools

import jax
import jax.numpy as jnp
from jax import lax
from jax.experimental import pallas as pl
from jax.experimental.pallas import tpu as pltpu
from jax.experimental.pallas import tpu_sc as plsc

D_MODEL = 768
BATCH = 4
SEQ = 2048
NB = BATCH * SEQ
NW = 32
S_PER_W = SEQ // NW
K = 16
N_CHUNKS = BATCH * S_PER_W // K
CPB = N_CHUNKS // BATCH
NBUF = 4
LANES = 16
JCOLS = D_MODEL // LANES


def _make_embed():
    mesh = plsc.VectorSubcoreMesh(core_axis_name="c", subcore_axis_name="s")

    @functools.partial(
        pl.kernel,
        mesh=mesh,
        out_type=jax.ShapeDtypeStruct((NB, D_MODEL), jnp.float32),
        scratch_types=[
            pltpu.VMEM((BATCH * S_PER_W,), jnp.int32),
            pltpu.VMEM((S_PER_W, D_MODEL), jnp.float32),
            pltpu.VMEM((NBUF, K, D_MODEL), jnp.float32),
            pltpu.SemaphoreType.DMA,
            pltpu.SemaphoreType.DMA,
            pltpu.SemaphoreType.DMA,
            pltpu.SemaphoreType.DMA,
        ],
    )
    def embed(x_hbm, table_hbm, pos_hbm, out_hbm,
              idx_v, pos_v, rows_v, isem, psem, gsem, ssem):
        wid = lax.axis_index("s") * 2 + lax.axis_index("c")
        s_base = pl.multiple_of(wid * S_PER_W, S_PER_W)

        idx_copies = [
            pltpu.async_copy(
                x_hbm.at[pl.ds(b * SEQ + s_base, S_PER_W)],
                idx_v.at[pl.ds(b * S_PER_W, S_PER_W)], isem)
            for b in range(BATCH)
        ]
        pos_copy = pltpu.async_copy(
            pos_hbm.at[pl.ds(s_base, S_PER_W)], pos_v, psem)
        for cp in idx_copies:
            cp.wait()

        def gather_desc(c):
            p = lax.bitwise_and(c, NBUF - 1)
            off = pl.multiple_of(lax.mul(c, K), 8)
            return pltpu.make_async_copy(
                table_hbm.at[idx_v.at[pl.ds(off, K)]], rows_v.at[p], gsem)

        def store_desc(c):
            p = lax.bitwise_and(c, NBUF - 1)
            b = lax.shift_right_logical(c, 2)
            h = lax.bitwise_and(c, CPB - 1)
            out_row = pl.multiple_of(b * SEQ + s_base + h * K, 8)
            return pltpu.make_async_copy(
                rows_v.at[p], out_hbm.at[pl.ds(out_row, K)], ssem)

        gather_desc(0).start()
        gather_desc(1).start()
        pos_copy.wait()

        def chunk(c, carry):
            @pl.when(c + 2 < N_CHUNKS)
            def _():
                @pl.when(c >= 2)
                def _():
                    store_desc(c - 2).wait()
                gather_desc(c + 2).start()

            gather_desc(c).wait()
            p = lax.bitwise_and(c, NBUF - 1)
            h = lax.bitwise_and(c, CPB - 1)

            @plsc.parallel_loop(0, K, 1, unroll=2)
            def row(r):
                for j in range(JCOLS):
                    sl = pl.ds(j * LANES, LANES)
                    plsc.addupdate(rows_v.at[p, r, sl], pos_v[h * K + r, sl])

            store_desc(c).start()
            return carry

        lax.fori_loop(0, N_CHUNKS, chunk, 0)
        for c in range(N_CHUNKS - NBUF, N_CHUNKS):
            store_desc(c).wait()

    return embed


_embed = _make_embed()


def kernel(x, token_table, pos_emb):
    x_flat = x.reshape(NB)
    pos = pos_emb.reshape(pos_emb.shape[1], D_MODEL)
    out = _embed(x_flat, token_table, pos)
    return out.reshape(BATCH, SEQ, D_MODEL)


# speedup vs baseline: 1.6580x; 1.0016x over previous
import functools

import jax
import jax.numpy as jnp
from jax import lax
from jax.experimental import pallas as pl
from jax.experimental.pallas import tpu as pltpu
from jax.experimental.pallas import tpu_sc as plsc

D_MODEL = 768
BATCH = 4
SEQ = 2048
NW = 32
S_PER_W = SEQ // NW
K = 16
N_CHUNKS = BATCH * S_PER_W // K
CPB = N_CHUNKS // BATCH
NBUF = 4
LANES = 16
JCOLS = D_MODEL // LANES


def _make_embed():
    mesh = plsc.VectorSubcoreMesh(core_axis_name="c", subcore_axis_name="s")

    @functools.partial(
        pl.kernel,
        mesh=mesh,
        out_type=jax.ShapeDtypeStruct((BATCH, SEQ, D_MODEL), jnp.float32),
        scratch_types=[
            pltpu.VMEM((BATCH * S_PER_W,), jnp.int32),
            pltpu.VMEM((S_PER_W, D_MODEL), jnp.float32),
            pltpu.VMEM((NBUF, K, D_MODEL), jnp.float32),
            pltpu.SemaphoreType.DMA,
            pltpu.SemaphoreType.DMA,
            pltpu.SemaphoreType.DMA,
            pltpu.SemaphoreType.DMA,
        ],
    )
    def embed(x_hbm, table_hbm, pos_hbm, out_hbm,
              idx_v, pos_v, rows_v, isem, psem, gsem, ssem):
        wid = lax.axis_index("s") * 2 + lax.axis_index("c")
        s_base = pl.multiple_of(wid * S_PER_W, S_PER_W)

        idx_copies = [
            pltpu.async_copy(
                x_hbm.at[b, pl.ds(s_base, S_PER_W)],
                idx_v.at[pl.ds(b * S_PER_W, S_PER_W)], isem)
            for b in range(BATCH)
        ]
        pos_copy = pltpu.async_copy(
            pos_hbm.at[0, pl.ds(s_base, S_PER_W)], pos_v, psem)
        for cp in idx_copies:
            cp.wait()

        def gather_desc(c):
            p = lax.bitwise_and(c, NBUF - 1)
            off = pl.multiple_of(lax.mul(c, K), 8)
            return pltpu.make_async_copy(
                table_hbm.at[idx_v.at[pl.ds(off, K)]], rows_v.at[p], gsem)

        def store_desc(c):
            p = lax.bitwise_and(c, NBUF - 1)
            b = lax.shift_right_logical(c, 2)
            h = lax.bitwise_and(c, CPB - 1)
            out_row = pl.multiple_of(s_base + h * K, 8)
            return pltpu.make_async_copy(
                rows_v.at[p], out_hbm.at[b, pl.ds(out_row, K)], ssem)

        gather_desc(0).start()
        gather_desc(1).start()
        pos_copy.wait()

        def chunk(c, carry):
            @pl.when(c + 2 < N_CHUNKS)
            def _():
                @pl.when(c >= 2)
                def _():
                    store_desc(c - 2).wait()
                gather_desc(c + 2).start()

            gather_desc(c).wait()
            p = lax.bitwise_and(c, NBUF - 1)
            h = lax.bitwise_and(c, CPB - 1)

            @plsc.parallel_loop(0, K, 1, unroll=2)
            def row(r):
                for j in range(JCOLS):
                    sl = pl.ds(j * LANES, LANES)
                    plsc.addupdate(rows_v.at[p, r, sl], pos_v[h * K + r, sl])

            store_desc(c).start()
            return carry

        lax.fori_loop(0, N_CHUNKS, chunk, 0)
        for c in range(N_CHUNKS - NBUF, N_CHUNKS):
            store_desc(c).wait()

    return embed


_embed = _make_embed()


def kernel(x, token_table, pos_emb):
    return _embed(x, token_table, pos_emb)
